# Initial kernel scaffold; baseline (speedup 1.0000x reference)
import jax
import jax.numpy as jnp
from jax.experimental import pallas as pl


def kernel(z):
    raise NotImplementedError("write your pallas kernel here")


# speedup vs baseline: 79.1181x; 79.1181x over previous
import jax
import jax.numpy as jnp
from jax.experimental import pallas as pl

_SOFT_DIM = 64


def _depth_norm_block(z_ref, out_ref):
    zb = z_ref[0, 0, :]
    z_norm = (jnp.clip(zb, -1.0, 1.0) + 1.0) / 2.0 * (_SOFT_DIM - 1)
    d = jax.lax.broadcasted_iota(
        jnp.int32, (_SOFT_DIM, zb.shape[0]), 0
    ).astype(jnp.float32)
    out_ref[0] = jnp.maximum(1.0 - jnp.abs(z_norm[None, :] - d), 0.0)


def kernel(z):
    B, _, N = z.shape
    nb = 8192
    out = pl.pallas_call(
        _depth_norm_block,
        grid=(B, N // nb),
        in_specs=[pl.BlockSpec((1, 1, nb), lambda b, n: (b, 0, n))],
        out_specs=pl.BlockSpec((1, _SOFT_DIM, nb), lambda b, n: (b, 0, n)),
        out_shape=jax.ShapeDtypeStruct((B, _SOFT_DIM, N), z.dtype),
    )(z)
    return out


# speedup vs baseline: 105.5225x; 1.3337x over previous
import jax
import jax.numpy as jnp
from jax.experimental import pallas as pl

_SOFT_DIM = 64


def _depth_norm_block(z_ref, out_ref):
    zb = z_ref[0, 0, :]
    z_norm = (jnp.clip(zb, -1.0, 1.0) + 1.0) / 2.0 * (_SOFT_DIM - 1)
    d = jax.lax.broadcasted_iota(
        jnp.int32, (_SOFT_DIM, zb.shape[0]), 0
    ).astype(jnp.float32)
    out_ref[0] = jnp.maximum(1.0 - jnp.abs(z_norm[None, :] - d), 0.0)


def kernel(z):
    B, _, N = z.shape
    nb = 16384
    out = pl.pallas_call(
        _depth_norm_block,
        grid=(B, N // nb),
        in_specs=[pl.BlockSpec((1, 1, nb), lambda b, n: (b, 0, n))],
        out_specs=pl.BlockSpec((1, _SOFT_DIM, nb), lambda b, n: (b, 0, n)),
        out_shape=jax.ShapeDtypeStruct((B, _SOFT_DIM, N), z.dtype),
    )(z)
    return out


# speedup vs baseline: 113.0147x; 1.0710x over previous
import jax
import jax.numpy as jnp
from jax.experimental import pallas as pl

_SOFT_DIM = 64


def _depth_norm_block(z_ref, out_ref):
    zb = z_ref[0, 0, :]
    z_norm = (jnp.clip(zb, -1.0, 1.0) + 1.0) / 2.0 * (_SOFT_DIM - 1)
    d = jax.lax.broadcasted_iota(
        jnp.int32, (_SOFT_DIM, zb.shape[0]), 0
    ).astype(jnp.float32)
    out_ref[0] = jnp.maximum(1.0 - jnp.abs(z_norm[None, :] - d), 0.0)


def kernel(z):
    B, _, N = z.shape
    nb = 32768
    out = pl.pallas_call(
        _depth_norm_block,
        grid=(B, N // nb),
        in_specs=[pl.BlockSpec((1, 1, nb), lambda b, n: (b, 0, n))],
        out_specs=pl.BlockSpec((1, _SOFT_DIM, nb), lambda b, n: (b, 0, n)),
        out_shape=jax.ShapeDtypeStruct((B, _SOFT_DIM, N), z.dtype),
    )(z)
    return out
